# Initial kernel scaffold; baseline (speedup 1.0000x reference)
import jax
import jax.numpy as jnp
from jax.experimental import pallas as pl


def kernel(x, edge_index, batch, emb, W1, b1, W2, b2, Wl, bl):
    raise NotImplementedError("write your pallas kernel here")


# speedup vs baseline: 6.1510x; 6.1510x over previous
import functools

import jax
import jax.numpy as jnp
from jax import lax
from jax.experimental import pallas as pl
from jax.experimental.pallas import tpu as pltpu
from jax.experimental.pallas import tpu_sc as plsc

N = 50000
E = 800000
VOCAB = 1000
G = 16
DE = 64
DH = 128
NCLS = 10

NP = 53248
EP = 802816
F = 32
RPT = NP // 16
RB = RPT // 128
EB_DEG = (EP // 32) // 128
EB_AGG = (EP // 16) // 128
R = 1024
GRID = NP // R

_mesh = plsc.VectorSubcoreMesh(core_axis_name="c", subcore_axis_name="s")
_sc_params = pltpu.CompilerParams(use_tc_tiling_on_sc=False)


def _t1_body(e_ref, w_ref, o_ref):
    o_ref[...] = jnp.dot(e_ref[...], w_ref[...], preferred_element_type=jnp.float32)


def _table(emb, W1):
    return pl.pallas_call(
        _t1_body,
        out_shape=jax.ShapeDtypeStruct((VOCAB, DH), jnp.float32),
    )(emb, W1)


def _dinv(d0, d1):
    return lax.rsqrt(d0[:, 0:1] + d1[:, 0:1] + 1.0)


def _scale_body(hw_ref, d0_ref, d1_ref, o0, o1, o2, o3):
    g = hw_ref[...] * _dinv(d0_ref[...], d1_ref[...])
    o0[...] = g[:, 0:32]
    o1[...] = g[:, 32:64]
    o2[...] = g[:, 64:96]
    o3[...] = g[:, 96:128]


def _scale_chunks(hw, d0, d1):
    return pl.pallas_call(
        _scale_body,
        grid=(GRID,),
        in_specs=[
            pl.BlockSpec((R, DH), lambda i: (i, 0)),
            pl.BlockSpec((R, 16), lambda i: (i, 0)),
            pl.BlockSpec((R, 16), lambda i: (i, 0)),
        ],
        out_specs=[pl.BlockSpec((R, F), lambda i: (i, 0))] * 4,
        out_shape=[jax.ShapeDtypeStruct((NP, F), jnp.float32)] * 4,
    )(hw, d0, d1)


def _mm_scale_body(h_ref, w_ref, d0_ref, d1_ref, o0, o1, o2, o3):
    hw = jnp.dot(h_ref[...], w_ref[...], preferred_element_type=jnp.float32)
    g = hw * _dinv(d0_ref[...], d1_ref[...])
    o0[...] = g[:, 0:32]
    o1[...] = g[:, 32:64]
    o2[...] = g[:, 64:96]
    o3[...] = g[:, 96:128]


def _mm_scale(h, W, d0, d1):
    return pl.pallas_call(
        _mm_scale_body,
        grid=(GRID,),
        in_specs=[
            pl.BlockSpec((R, DH), lambda i: (i, 0)),
            pl.BlockSpec((DH, DH), lambda i: (0, 0)),
            pl.BlockSpec((R, 16), lambda i: (i, 0)),
            pl.BlockSpec((R, 16), lambda i: (i, 0)),
        ],
        out_specs=[pl.BlockSpec((R, F), lambda i: (i, 0))] * 4,
        out_shape=[jax.ShapeDtypeStruct((NP, F), jnp.float32)] * 4,
    )(h, W, d0, d1)


def _post_body(a0, a1, a2, a3, g0, g1, g2, g3, d0_ref, d1_ref, b_ref, o_ref):
    agg = jnp.concatenate([a0[...], a1[...], a2[...], a3[...]], axis=1)
    gg = jnp.concatenate([g0[...], g1[...], g2[...], g3[...]], axis=1)
    h = (agg + gg) * _dinv(d0_ref[...], d1_ref[...]) + b_ref[...]
    o_ref[...] = jnp.maximum(h, 0.0)


def _post(aggs, gs, d0, d1, b):
    return pl.pallas_call(
        _post_body,
        grid=(GRID,),
        in_specs=[pl.BlockSpec((R, F), lambda i: (i, 0))] * 8 + [
            pl.BlockSpec((R, 16), lambda i: (i, 0)),
            pl.BlockSpec((R, 16), lambda i: (i, 0)),
            pl.BlockSpec((1, DH), lambda i: (0, 0)),
        ],
        out_specs=pl.BlockSpec((R, DH), lambda i: (i, 0)),
        out_shape=jax.ShapeDtypeStruct((NP, DH), jnp.float32),
    )(*aggs, *gs, d0, d1, b)


def _final_body(a0, a1, a2, a3, g0, g1, g2, g3, d0_ref, d1_ref, b_ref,
                bat_ref, wl_ref, bl_ref, o_ref, sums, cnt):
    i = pl.program_id(0)
    agg = jnp.concatenate([a0[...], a1[...], a2[...], a3[...]], axis=1)
    gg = jnp.concatenate([g0[...], g1[...], g2[...], g3[...]], axis=1)
    h = (agg + gg) * _dinv(d0_ref[...], d1_ref[...]) + b_ref[...]
    h = jnp.maximum(h, 0.0)
    oh = (bat_ref[...] == lax.broadcasted_iota(jnp.int32, (R, G), 1)
          ).astype(jnp.float32)
    dn = (((0,), (0,)), ((), ()))
    ps = lax.dot_general(oh, h, dn, preferred_element_type=jnp.float32)
    pc = lax.dot_general(oh, jnp.ones_like(h), dn,
                         preferred_element_type=jnp.float32)

    @pl.when(i == 0)
    def _init():
        sums[...] = ps
        cnt[...] = pc

    @pl.when(i > 0)
    def _acc():
        sums[...] += ps
        cnt[...] += pc

    @pl.when(i == GRID - 1)
    def _fin():
        pooled = sums[...] / jnp.maximum(cnt[...], 1.0)
        o_ref[...] = jnp.dot(pooled, wl_ref[...],
                             preferred_element_type=jnp.float32) + bl_ref[...]


def _final(aggs, gs, d0, d1, b, bat, Wl, bl):
    return pl.pallas_call(
        _final_body,
        grid=(GRID,),
        in_specs=[pl.BlockSpec((R, F), lambda i: (i, 0))] * 8 + [
            pl.BlockSpec((R, 16), lambda i: (i, 0)),
            pl.BlockSpec((R, 16), lambda i: (i, 0)),
            pl.BlockSpec((1, DH), lambda i: (0, 0)),
            pl.BlockSpec((R, 1), lambda i: (i, 0)),
            pl.BlockSpec((DH, NCLS), lambda i: (0, 0)),
            pl.BlockSpec((1, NCLS), lambda i: (0, 0)),
        ],
        out_specs=pl.BlockSpec((G, NCLS), lambda i: (0, 0)),
        out_shape=jax.ShapeDtypeStruct((G, NCLS), jnp.float32),
        scratch_shapes=[
            pltpu.VMEM((G, DH), jnp.float32),
            pltpu.VMEM((G, DH), jnp.float32),
        ],
    )(*aggs, *gs, d0, d1, b, bat, Wl, bl)


@functools.partial(
    pl.kernel, mesh=_mesh,
    out_type=jax.ShapeDtypeStruct((NP, DH), jnp.float32),
    scratch_types=[
        pltpu.VMEM((128,), jnp.int32),
        pltpu.VMEM((128, DH), jnp.float32),
        pltpu.SemaphoreType.DMA,
    ],
    compiler_params=_sc_params,
)
def _sc_gather(tab, xi, o, idx, buf, sem):
    c = lax.axis_index("c")
    s = lax.axis_index("s")
    wid = s * 2 + c

    def body(k, carry):
        base = wid * (NP // 32) + k * 128
        pltpu.sync_copy(xi.at[pl.ds(base, 128)], idx)
        pltpu.async_copy(tab.at[idx], buf, sem).wait()
        pltpu.sync_copy(buf, o.at[pl.ds(base, 128)])
        return carry

    lax.fori_loop(0, NP // 32 // 128, body, 0)


@functools.partial(
    pl.kernel, mesh=_mesh,
    out_type=jax.ShapeDtypeStruct((2, NP, 16), jnp.float32),
    scratch_types=[
        pltpu.VMEM((128,), jnp.int32),
        pltpu.VMEM((128, 16), jnp.float32),
        pltpu.VMEM((128, 16), jnp.float32),
        pltpu.VMEM_SHARED((NP, 16), jnp.float32),
    ],
    compiler_params=_sc_params,
)
def _sc_deg(dst, o, idx, ones_b, zb, acc):
    c = lax.axis_index("c")
    s = lax.axis_index("s")
    for i in range(128):
        ones_b[i, pl.ds(0, 16)] = jnp.ones((16,), jnp.float32)
        zb[i, pl.ds(0, 16)] = jnp.zeros((16,), jnp.float32)

    def zbody(k, carry):
        pltpu.sync_copy(zb, acc.at[pl.ds(s * RPT + k * 128, 128)])
        return carry

    lax.fori_loop(0, RB, zbody, 0)
    plsc.subcore_barrier()

    def ebody(k, carry):
        base = c * (EP // 2) + s * (EP // 32) + k * 128
        pltpu.sync_copy(dst.at[pl.ds(base, 128)], idx)
        pltpu.sync_copy(ones_b, acc.at[idx], add=True)
        return carry

    lax.fori_loop(0, EB_DEG, ebody, 0)
    plsc.subcore_barrier()
    pltpu.sync_copy(acc.at[pl.ds(s * RPT, RPT)], o.at[c, pl.ds(s * RPT, RPT)])


@functools.partial(
    pl.kernel, mesh=_mesh,
    out_type=[jax.ShapeDtypeStruct((NP, F), jnp.float32)] * 4,
    scratch_types=[
        pltpu.VMEM((128,), jnp.int32),
        pltpu.VMEM((128,), jnp.int32),
        pltpu.VMEM((128, F), jnp.float32),
        pltpu.VMEM((128, F), jnp.float32),
        pltpu.VMEM_SHARED((NP, F), jnp.float32),
        pltpu.SemaphoreType.DMA,
    ],
    compiler_params=_sc_params,
)
def _sc_agg(g0, g1, g2, g3, src, dst, o0, o1, o2, o3,
            idx_s, idx_d, buf, zb, acc, sem):
    c = lax.axis_index("c")
    s = lax.axis_index("s")
    for i in range(128):
        for j in range(F // 16):
            zb[i, pl.ds(j * 16, 16)] = jnp.zeros((16,), jnp.float32)

    def run(g, o):
        def zbody(k, carry):
            pltpu.sync_copy(zb, acc.at[pl.ds(s * RPT + k * 128, 128)])
            return carry

        lax.fori_loop(0, RB, zbody, 0)
        plsc.subcore_barrier()

        def ebody(k, carry):
            base = s * (EP // 16) + k * 128
            pltpu.sync_copy(src.at[pl.ds(base, 128)], idx_s)
            pltpu.sync_copy(dst.at[pl.ds(base, 128)], idx_d)
            pltpu.async_copy(g.at[idx_s], buf, sem).wait()
            pltpu.sync_copy(buf, acc.at[idx_d], add=True)
            return carry

        lax.fori_loop(0, EB_AGG, ebody, 0)
        plsc.subcore_barrier()
        pltpu.sync_copy(acc.at[pl.ds(s * RPT, RPT)], o.at[pl.ds(s * RPT, RPT)])
        plsc.subcore_barrier()

    @pl.when(c == 0)
    def _sc0():
        run(g0, o0)
        run(g1, o1)

    @pl.when(c == 1)
    def _sc1():
        run(g2, o2)
        run(g3, o3)


def kernel(x, edge_index, batch, emb, W1, b1, W2, b2, Wl, bl):
    x_pad = jnp.concatenate([x, jnp.zeros((NP - N,), jnp.int32)])
    src = jnp.concatenate([edge_index[0], jnp.full((EP - E,), N, jnp.int32)])
    dst = jnp.concatenate([edge_index[1], jnp.full((EP - E,), N, jnp.int32)])
    bat = jnp.concatenate([batch, jnp.full((NP - N,), G, jnp.int32)])
    bat = bat.reshape(NP, 1)
    b1r = b1.reshape(1, DH)
    b2r = b2.reshape(1, DH)
    blr = bl.reshape(1, NCLS)

    T1 = _table(emb, W1)
    hw1 = _sc_gather(T1, x_pad)
    dg = _sc_deg(dst)
    d0, d1 = dg[0], dg[1]

    g1 = _scale_chunks(hw1, d0, d1)
    a1 = _sc_agg(*g1, src, dst)
    h1 = _post(a1, g1, d0, d1, b1r)

    g2 = _mm_scale(h1, W2, d0, d1)
    a2 = _sc_agg(*g2, src, dst)
    return _final(a2, g2, d0, d1, b2r, bat, Wl, blr)


# speedup vs baseline: 11.7193x; 1.9053x over previous
import functools

import jax
import jax.numpy as jnp
from jax import lax
from jax.experimental import pallas as pl
from jax.experimental.pallas import tpu as pltpu
from jax.experimental.pallas import tpu_sc as plsc

N = 50000
E = 800000
VOCAB = 1000
G = 16
DE = 64
DH = 128
NCLS = 10

NP = 53248
EP = 802816
F = 32
RPT = NP // 16
RB = RPT // 128
EB_DEG = (EP // 32) // 128
EB_AGG = (EP // 16) // 128
R = 1024
GRID = NP // R

_mesh = plsc.VectorSubcoreMesh(core_axis_name="c", subcore_axis_name="s")
_sc_params = pltpu.CompilerParams(use_tc_tiling_on_sc=False)


def _t1_body(e_ref, w_ref, o_ref):
    o_ref[...] = jnp.dot(e_ref[...], w_ref[...], preferred_element_type=jnp.float32)


def _table(emb, W1):
    return pl.pallas_call(
        _t1_body,
        out_shape=jax.ShapeDtypeStruct((VOCAB, DH), jnp.float32),
    )(emb, W1)


def _dinv(d0, d1):
    return lax.rsqrt(d0[:, 0:1] + d1[:, 0:1] + 1.0)


def _scale_body(hw_ref, d0_ref, d1_ref, o0, o1, o2, o3):
    g = hw_ref[...] * _dinv(d0_ref[...], d1_ref[...])
    o0[...] = g[:, 0:32]
    o1[...] = g[:, 32:64]
    o2[...] = g[:, 64:96]
    o3[...] = g[:, 96:128]


def _scale_chunks(hw, d0, d1):
    return pl.pallas_call(
        _scale_body,
        grid=(GRID,),
        in_specs=[
            pl.BlockSpec((R, DH), lambda i: (i, 0)),
            pl.BlockSpec((R, 16), lambda i: (i, 0)),
            pl.BlockSpec((R, 16), lambda i: (i, 0)),
        ],
        out_specs=[pl.BlockSpec((R, F), lambda i: (i, 0))] * 4,
        out_shape=[jax.ShapeDtypeStruct((NP, F), jnp.float32)] * 4,
    )(hw, d0, d1)


def _mm_scale_body(h_ref, w_ref, d0_ref, d1_ref, o0, o1, o2, o3):
    hw = jnp.dot(h_ref[...], w_ref[...], preferred_element_type=jnp.float32)
    g = hw * _dinv(d0_ref[...], d1_ref[...])
    o0[...] = g[:, 0:32]
    o1[...] = g[:, 32:64]
    o2[...] = g[:, 64:96]
    o3[...] = g[:, 96:128]


def _mm_scale(h, W, d0, d1):
    return pl.pallas_call(
        _mm_scale_body,
        grid=(GRID,),
        in_specs=[
            pl.BlockSpec((R, DH), lambda i: (i, 0)),
            pl.BlockSpec((DH, DH), lambda i: (0, 0)),
            pl.BlockSpec((R, 16), lambda i: (i, 0)),
            pl.BlockSpec((R, 16), lambda i: (i, 0)),
        ],
        out_specs=[pl.BlockSpec((R, F), lambda i: (i, 0))] * 4,
        out_shape=[jax.ShapeDtypeStruct((NP, F), jnp.float32)] * 4,
    )(h, W, d0, d1)


def _post_body(a0, a1, a2, a3, g0, g1, g2, g3, d0_ref, d1_ref, b_ref, o_ref):
    agg = jnp.concatenate([a0[...], a1[...], a2[...], a3[...]], axis=1)
    gg = jnp.concatenate([g0[...], g1[...], g2[...], g3[...]], axis=1)
    h = (agg + gg) * _dinv(d0_ref[...], d1_ref[...]) + b_ref[...]
    o_ref[...] = jnp.maximum(h, 0.0)


def _post(aggs, gs, d0, d1, b):
    return pl.pallas_call(
        _post_body,
        grid=(GRID,),
        in_specs=[pl.BlockSpec((R, F), lambda i: (i, 0))] * 8 + [
            pl.BlockSpec((R, 16), lambda i: (i, 0)),
            pl.BlockSpec((R, 16), lambda i: (i, 0)),
            pl.BlockSpec((1, DH), lambda i: (0, 0)),
        ],
        out_specs=pl.BlockSpec((R, DH), lambda i: (i, 0)),
        out_shape=jax.ShapeDtypeStruct((NP, DH), jnp.float32),
    )(*aggs, *gs, d0, d1, b)


def _final_body(a0, a1, a2, a3, g0, g1, g2, g3, d0_ref, d1_ref, b_ref,
                bat_ref, wl_ref, bl_ref, o_ref, sums, cnt):
    i = pl.program_id(0)
    agg = jnp.concatenate([a0[...], a1[...], a2[...], a3[...]], axis=1)
    gg = jnp.concatenate([g0[...], g1[...], g2[...], g3[...]], axis=1)
    h = (agg + gg) * _dinv(d0_ref[...], d1_ref[...]) + b_ref[...]
    h = jnp.maximum(h, 0.0)
    oh = (bat_ref[...] == lax.broadcasted_iota(jnp.int32, (R, G), 1)
          ).astype(jnp.float32)
    dn = (((0,), (0,)), ((), ()))
    ps = lax.dot_general(oh, h, dn, preferred_element_type=jnp.float32)
    pc = lax.dot_general(oh, jnp.ones_like(h), dn,
                         preferred_element_type=jnp.float32)

    @pl.when(i == 0)
    def _init():
        sums[...] = ps
        cnt[...] = pc

    @pl.when(i > 0)
    def _acc():
        sums[...] += ps
        cnt[...] += pc

    @pl.when(i == GRID - 1)
    def _fin():
        pooled = sums[...] / jnp.maximum(cnt[...], 1.0)
        o_ref[...] = jnp.dot(pooled, wl_ref[...],
                             preferred_element_type=jnp.float32) + bl_ref[...]


def _final(aggs, gs, d0, d1, b, bat, Wl, bl):
    return pl.pallas_call(
        _final_body,
        grid=(GRID,),
        in_specs=[pl.BlockSpec((R, F), lambda i: (i, 0))] * 8 + [
            pl.BlockSpec((R, 16), lambda i: (i, 0)),
            pl.BlockSpec((R, 16), lambda i: (i, 0)),
            pl.BlockSpec((1, DH), lambda i: (0, 0)),
            pl.BlockSpec((R, 1), lambda i: (i, 0)),
            pl.BlockSpec((DH, NCLS), lambda i: (0, 0)),
            pl.BlockSpec((1, NCLS), lambda i: (0, 0)),
        ],
        out_specs=pl.BlockSpec((G, NCLS), lambda i: (0, 0)),
        out_shape=jax.ShapeDtypeStruct((G, NCLS), jnp.float32),
        scratch_shapes=[
            pltpu.VMEM((G, DH), jnp.float32),
            pltpu.VMEM((G, DH), jnp.float32),
        ],
    )(*aggs, *gs, d0, d1, b, bat, Wl, bl)


@functools.partial(
    pl.kernel, mesh=_mesh,
    out_type=jax.ShapeDtypeStruct((NP, DH), jnp.float32),
    scratch_types=[
        pltpu.VMEM((13, 128), jnp.int32),
        pltpu.VMEM((4, 128, DH), jnp.float32),
        pltpu.SemaphoreType.DMA,
    ],
    compiler_params=_sc_params,
)
def _sc_gather(tab, xi, o, idx, buf, sem):
    c = lax.axis_index("c")
    s = lax.axis_index("s")
    wid = s * 2 + c
    nb = NP // 32 // 128
    row0 = wid * nb
    pltpu.sync_copy(xi.at[pl.ds(row0, nb)], idx)
    descs = [None] * nb
    for j in range(nb):
        if j >= 4:
            descs[j - 4].wait()
            pltpu.sync_copy(buf.at[(j - 4) % 4],
                            o.at[pl.ds((row0 + j - 4) * 128, 128)])
        descs[j] = pltpu.async_copy(tab.at[idx.at[j]], buf.at[j % 4], sem)
    for j in range(nb - 4, nb):
        descs[j].wait()
        pltpu.sync_copy(buf.at[j % 4], o.at[pl.ds((row0 + j) * 128, 128)])


@functools.partial(
    pl.kernel, mesh=_mesh,
    out_type=jax.ShapeDtypeStruct((2, NP, 16), jnp.float32),
    scratch_types=[
        pltpu.VMEM((4, 128), jnp.int32),
        pltpu.VMEM((128, 16), jnp.float32),
        pltpu.VMEM((128, 16), jnp.float32),
        pltpu.VMEM_SHARED((NP, 16), jnp.float32),
        pltpu.SemaphoreType.DMA,
    ],
    compiler_params=_sc_params,
)
def _sc_deg(dst, o, idx, ones_b, zb, acc, sem):
    c = lax.axis_index("c")
    s = lax.axis_index("s")
    for i in range(128):
        ones_b[i, pl.ds(0, 16)] = jnp.ones((16,), jnp.float32)
        zb[i, pl.ds(0, 16)] = jnp.zeros((16,), jnp.float32)

    def zbody(k, carry):
        pltpu.sync_copy(zb, acc.at[pl.ds(s * RPT + k * 128, 128)])
        return carry

    lax.fori_loop(0, RB, zbody, 0)
    plsc.subcore_barrier()

    def ebody(k, carry):
        row0 = c * (EP // 256) + s * EB_DEG + k * 4
        pltpu.sync_copy(dst.at[pl.ds(row0, 4)], idx)
        descs = [pltpu.async_copy(ones_b, acc.at[idx.at[j]], sem, add=True)
                 for j in range(4)]
        for d in descs:
            d.wait()
        return carry

    lax.fori_loop(0, EB_DEG // 4, ebody, 0)
    plsc.subcore_barrier()
    pltpu.sync_copy(acc.at[pl.ds(s * RPT, RPT)], o.at[c, pl.ds(s * RPT, RPT)])


@functools.partial(
    pl.kernel, mesh=_mesh,
    out_type=[jax.ShapeDtypeStruct((NP, F), jnp.float32)] * 4,
    scratch_types=[
        pltpu.VMEM((4, 128), jnp.int32),
        pltpu.VMEM((4, 128), jnp.int32),
        pltpu.VMEM((4, 128, F), jnp.float32),
        pltpu.VMEM((128, F), jnp.float32),
        pltpu.VMEM_SHARED((NP, F), jnp.float32),
        pltpu.SemaphoreType.DMA,
        pltpu.SemaphoreType.DMA,
    ],
    compiler_params=_sc_params,
)
def _sc_agg(g0, g1, g2, g3, src, dst, o0, o1, o2, o3,
            idx_s, idx_d, buf, zb, acc, sem_g, sem_s):
    c = lax.axis_index("c")
    s = lax.axis_index("s")
    for i in range(128):
        for j in range(F // 16):
            zb[i, pl.ds(j * 16, 16)] = jnp.zeros((16,), jnp.float32)

    def run(g, o):
        def zbody(k, carry):
            pltpu.sync_copy(zb, acc.at[pl.ds(s * RPT + k * 128, 128)])
            return carry

        lax.fori_loop(0, RB, zbody, 0)
        plsc.subcore_barrier()

        def ebody(k, carry):
            row0 = s * EB_AGG + k * 4
            pltpu.sync_copy(src.at[pl.ds(row0, 4)], idx_s)
            pltpu.sync_copy(dst.at[pl.ds(row0, 4)], idx_d)
            gd = [pltpu.async_copy(g.at[idx_s.at[j]], buf.at[j], sem_g)
                  for j in range(4)]
            sd = [None] * 4
            for j in range(4):
                gd[j].wait()
                sd[j] = pltpu.async_copy(buf.at[j], acc.at[idx_d.at[j]],
                                         sem_s, add=True)
            for j in range(4):
                sd[j].wait()
            return carry

        lax.fori_loop(0, EB_AGG // 4, ebody, 0)
        plsc.subcore_barrier()
        pltpu.sync_copy(acc.at[pl.ds(s * RPT, RPT)], o.at[pl.ds(s * RPT, RPT)])
        plsc.subcore_barrier()

    @pl.when(c == 0)
    def _sc0():
        run(g0, o0)
        run(g1, o1)

    @pl.when(c == 1)
    def _sc1():
        run(g2, o2)
        run(g3, o3)


def kernel(x, edge_index, batch, emb, W1, b1, W2, b2, Wl, bl):
    x_pad = jnp.concatenate([x, jnp.zeros((NP - N,), jnp.int32)])
    x_pad = x_pad.reshape(NP // 128, 128)
    src = jnp.concatenate([edge_index[0], jnp.full((EP - E,), N, jnp.int32)])
    dst = jnp.concatenate([edge_index[1], jnp.full((EP - E,), N, jnp.int32)])
    src = src.reshape(EP // 128, 128)
    dst = dst.reshape(EP // 128, 128)
    bat = jnp.concatenate([batch, jnp.full((NP - N,), G, jnp.int32)])
    bat = bat.reshape(NP, 1)
    b1r = b1.reshape(1, DH)
    b2r = b2.reshape(1, DH)
    blr = bl.reshape(1, NCLS)

    T1 = _table(emb, W1)
    hw1 = _sc_gather(T1, x_pad)
    dg = _sc_deg(dst)
    d0, d1 = dg[0], dg[1]

    g1 = _scale_chunks(hw1, d0, d1)
    a1 = _sc_agg(*g1, src, dst)
    h1 = _post(a1, g1, d0, d1, b1r)

    g2 = _mm_scale(h1, W2, d0, d1)
    a2 = _sc_agg(*g2, src, dst)
    return _final(a2, g2, d0, d1, b2r, bat, Wl, blr)


# speedup vs baseline: 12.9937x; 1.1087x over previous
import functools

import jax
import jax.numpy as jnp
from jax import lax
from jax.experimental import pallas as pl
from jax.experimental.pallas import tpu as pltpu
from jax.experimental.pallas import tpu_sc as plsc

N = 50000
E = 800000
VOCAB = 1000
G = 16
DE = 64
DH = 128
NCLS = 10

NP = 53248
EP = 802816
F = 32
RPT = NP // 16
RB = RPT // 128
EB_DEG = (EP // 32) // 128
EB_AGG = (EP // 16) // 128
R = 1024
GRID = NP // R

_mesh = plsc.VectorSubcoreMesh(core_axis_name="c", subcore_axis_name="s")
_sc_params = pltpu.CompilerParams(use_tc_tiling_on_sc=False)


def _t1_body(e_ref, w_ref, o_ref):
    o_ref[...] = jnp.dot(e_ref[...], w_ref[...], preferred_element_type=jnp.float32)


def _table(emb, W1):
    return pl.pallas_call(
        _t1_body,
        out_shape=jax.ShapeDtypeStruct((VOCAB, DH), jnp.float32),
    )(emb, W1)


def _dinv(d0, d1):
    return lax.rsqrt(d0[:, 0:1] + d1[:, 0:1] + 1.0)


def _scale_body(hw_ref, d0_ref, d1_ref, o0, o1, o2, o3):
    g = hw_ref[...] * _dinv(d0_ref[...], d1_ref[...])
    o0[...] = g[:, 0:32]
    o1[...] = g[:, 32:64]
    o2[...] = g[:, 64:96]
    o3[...] = g[:, 96:128]


def _scale_chunks(hw, d0, d1):
    return pl.pallas_call(
        _scale_body,
        grid=(GRID,),
        in_specs=[
            pl.BlockSpec((R, DH), lambda i: (i, 0)),
            pl.BlockSpec((R, 16), lambda i: (i, 0)),
            pl.BlockSpec((R, 16), lambda i: (i, 0)),
        ],
        out_specs=[pl.BlockSpec((R, F), lambda i: (i, 0))] * 4,
        out_shape=[jax.ShapeDtypeStruct((NP, F), jnp.float32)] * 4,
    )(hw, d0, d1)


def _post_mm_body(a0, a1, a2, a3, g0, g1, g2, g3, d0_ref, d1_ref, b_ref,
                  w_ref, o0, o1, o2, o3):
    agg = jnp.concatenate([a0[...], a1[...], a2[...], a3[...]], axis=1)
    gg = jnp.concatenate([g0[...], g1[...], g2[...], g3[...]], axis=1)
    dinv = _dinv(d0_ref[...], d1_ref[...])
    h = jnp.maximum((agg + gg) * dinv + b_ref[...], 0.0)
    g = jnp.dot(h, w_ref[...], preferred_element_type=jnp.float32) * dinv
    o0[...] = g[:, 0:32]
    o1[...] = g[:, 32:64]
    o2[...] = g[:, 64:96]
    o3[...] = g[:, 96:128]


def _post_mm(aggs, gs, d0, d1, b, W):
    return pl.pallas_call(
        _post_mm_body,
        grid=(GRID,),
        in_specs=[pl.BlockSpec((R, F), lambda i: (i, 0))] * 8 + [
            pl.BlockSpec((R, 16), lambda i: (i, 0)),
            pl.BlockSpec((R, 16), lambda i: (i, 0)),
            pl.BlockSpec((1, DH), lambda i: (0, 0)),
            pl.BlockSpec((DH, DH), lambda i: (0, 0)),
        ],
        out_specs=[pl.BlockSpec((R, F), lambda i: (i, 0))] * 4,
        out_shape=[jax.ShapeDtypeStruct((NP, F), jnp.float32)] * 4,
    )(*aggs, *gs, d0, d1, b, W)


def _final_body(a0, a1, a2, a3, g0, g1, g2, g3, d0_ref, d1_ref, b_ref,
                bat_ref, wl_ref, bl_ref, o_ref, sums, cnt):
    i = pl.program_id(0)
    agg = jnp.concatenate([a0[...], a1[...], a2[...], a3[...]], axis=1)
    gg = jnp.concatenate([g0[...], g1[...], g2[...], g3[...]], axis=1)
    h = (agg + gg) * _dinv(d0_ref[...], d1_ref[...]) + b_ref[...]
    h = jnp.maximum(h, 0.0)
    oh = (bat_ref[...] == lax.broadcasted_iota(jnp.int32, (R, G), 1)
          ).astype(jnp.float32)
    dn = (((0,), (0,)), ((), ()))
    ps = lax.dot_general(oh, h, dn, preferred_element_type=jnp.float32)
    pc = lax.dot_general(oh, jnp.ones_like(h), dn,
                         preferred_element_type=jnp.float32)

    @pl.when(i == 0)
    def _init():
        sums[...] = ps
        cnt[...] = pc

    @pl.when(i > 0)
    def _acc():
        sums[...] += ps
        cnt[...] += pc

    @pl.when(i == GRID - 1)
    def _fin():
        pooled = sums[...] / jnp.maximum(cnt[...], 1.0)
        o_ref[...] = jnp.dot(pooled, wl_ref[...],
                             preferred_element_type=jnp.float32) + bl_ref[...]


def _final(aggs, gs, d0, d1, b, bat, Wl, bl):
    return pl.pallas_call(
        _final_body,
        grid=(GRID,),
        in_specs=[pl.BlockSpec((R, F), lambda i: (i, 0))] * 8 + [
            pl.BlockSpec((R, 16), lambda i: (i, 0)),
            pl.BlockSpec((R, 16), lambda i: (i, 0)),
            pl.BlockSpec((1, DH), lambda i: (0, 0)),
            pl.BlockSpec((R, 1), lambda i: (i, 0)),
            pl.BlockSpec((DH, NCLS), lambda i: (0, 0)),
            pl.BlockSpec((1, NCLS), lambda i: (0, 0)),
        ],
        out_specs=pl.BlockSpec((G, NCLS), lambda i: (0, 0)),
        out_shape=jax.ShapeDtypeStruct((G, NCLS), jnp.float32),
        scratch_shapes=[
            pltpu.VMEM((G, DH), jnp.float32),
            pltpu.VMEM((G, DH), jnp.float32),
        ],
    )(*aggs, *gs, d0, d1, b, bat, Wl, bl)


@functools.partial(
    pl.kernel, mesh=_mesh,
    out_type=jax.ShapeDtypeStruct((NP, DH), jnp.float32),
    scratch_types=[
        pltpu.VMEM((13, 128), jnp.int32),
        pltpu.VMEM((4, 128, DH), jnp.float32),
        pltpu.SemaphoreType.DMA,
    ],
    compiler_params=_sc_params,
)
def _sc_gather(tab, xi, o, idx, buf, sem):
    c = lax.axis_index("c")
    s = lax.axis_index("s")
    wid = s * 2 + c
    nb = NP // 32 // 128
    row0 = wid * nb
    pltpu.sync_copy(xi.at[pl.ds(row0, nb)], idx)
    descs = [None] * nb
    for j in range(nb):
        if j >= 4:
            descs[j - 4].wait()
            pltpu.sync_copy(buf.at[(j - 4) % 4],
                            o.at[pl.ds((row0 + j - 4) * 128, 128)])
        descs[j] = pltpu.async_copy(tab.at[idx.at[j]], buf.at[j % 4], sem)
    for j in range(nb - 4, nb):
        descs[j].wait()
        pltpu.sync_copy(buf.at[j % 4], o.at[pl.ds((row0 + j) * 128, 128)])


@functools.partial(
    pl.kernel, mesh=_mesh,
    out_type=jax.ShapeDtypeStruct((2, NP, 16), jnp.float32),
    scratch_types=[
        pltpu.VMEM((4, 128), jnp.int32),
        pltpu.VMEM((128, 16), jnp.float32),
        pltpu.VMEM((128, 16), jnp.float32),
        pltpu.VMEM_SHARED((NP, 16), jnp.float32),
        pltpu.SemaphoreType.DMA,
    ],
    compiler_params=_sc_params,
)
def _sc_deg(dst, o, idx, ones_b, zb, acc, sem):
    c = lax.axis_index("c")
    s = lax.axis_index("s")
    for i in range(128):
        ones_b[i, pl.ds(0, 16)] = jnp.ones((16,), jnp.float32)
        zb[i, pl.ds(0, 16)] = jnp.zeros((16,), jnp.float32)

    def zbody(k, carry):
        pltpu.sync_copy(zb, acc.at[pl.ds(s * RPT + k * 128, 128)])
        return carry

    lax.fori_loop(0, RB, zbody, 0)
    plsc.subcore_barrier()

    def ebody(k, carry):
        row0 = c * (EP // 256) + s * EB_DEG + k * 4
        pltpu.sync_copy(dst.at[pl.ds(row0, 4)], idx)
        descs = [pltpu.async_copy(ones_b, acc.at[idx.at[j]], sem, add=True)
                 for j in range(4)]
        for d in descs:
            d.wait()
        return carry

    lax.fori_loop(0, EB_DEG // 4, ebody, 0)
    plsc.subcore_barrier()
    pltpu.sync_copy(acc.at[pl.ds(s * RPT, RPT)], o.at[c, pl.ds(s * RPT, RPT)])


@functools.partial(
    pl.kernel, mesh=_mesh,
    out_type=[jax.ShapeDtypeStruct((NP, F), jnp.float32)] * 4,
    scratch_types=[
        pltpu.VMEM((8, 128), jnp.int32),
        pltpu.VMEM((4, 128, F), jnp.float32),
        pltpu.VMEM((128, F), jnp.float32),
        pltpu.VMEM_SHARED((NP, F), jnp.float32),
        pltpu.SemaphoreType.DMA,
        pltpu.SemaphoreType.DMA,
    ],
    compiler_params=_sc_params,
)
def _sc_agg(g0, g1, g2, g3, sd_idx, o0, o1, o2, o3,
            idx8, buf, zb, acc, sem_g, sem_s):
    c = lax.axis_index("c")
    s = lax.axis_index("s")
    for i in range(128):
        for j in range(F // 16):
            zb[i, pl.ds(j * 16, 16)] = jnp.zeros((16,), jnp.float32)

    def run(g, o):
        def zbody(k, carry):
            pltpu.sync_copy(zb, acc.at[pl.ds(s * RPT + k * 128, 128)])
            return carry

        lax.fori_loop(0, RB, zbody, 0)
        plsc.subcore_barrier()

        def ebody(k, carry):
            grp = s * (EB_AGG // 4) + k
            pltpu.sync_copy(sd_idx.at[grp], idx8)
            gd = [pltpu.async_copy(g.at[idx8.at[j]], buf.at[j], sem_g)
                  for j in range(4)]
            sd = [None] * 4
            for j in range(4):
                gd[j].wait()
                sd[j] = pltpu.async_copy(buf.at[j], acc.at[idx8.at[4 + j]],
                                         sem_s, add=True)
            for j in range(4):
                sd[j].wait()
            return carry

        lax.fori_loop(0, EB_AGG // 4, ebody, 0)
        plsc.subcore_barrier()
        pltpu.sync_copy(acc.at[pl.ds(s * RPT, RPT)], o.at[pl.ds(s * RPT, RPT)])
        plsc.subcore_barrier()

    @pl.when(c == 0)
    def _sc0():
        run(g0, o0)
        run(g1, o1)

    @pl.when(c == 1)
    def _sc1():
        run(g2, o2)
        run(g3, o3)


def kernel(x, edge_index, batch, emb, W1, b1, W2, b2, Wl, bl):
    x_pad = jnp.concatenate([x, jnp.zeros((NP - N,), jnp.int32)])
    x_pad = x_pad.reshape(NP // 128, 128)
    src = jnp.concatenate([edge_index[0], jnp.full((EP - E,), N, jnp.int32)])
    dst = jnp.concatenate([edge_index[1], jnp.full((EP - E,), N, jnp.int32)])
    sd_idx = jnp.concatenate([src.reshape(EP // 512, 4, 128),
                              dst.reshape(EP // 512, 4, 128)], axis=1)
    dst = dst.reshape(EP // 128, 128)
    bat = jnp.concatenate([batch, jnp.full((NP - N,), G, jnp.int32)])
    bat = bat.reshape(NP, 1)
    b1r = b1.reshape(1, DH)
    b2r = b2.reshape(1, DH)
    blr = bl.reshape(1, NCLS)

    T1 = _table(emb, W1)
    hw1 = _sc_gather(T1, x_pad)
    dg = _sc_deg(dst)
    d0, d1 = dg[0], dg[1]

    g1 = _scale_chunks(hw1, d0, d1)
    a1 = _sc_agg(*g1, sd_idx)
    g2 = _post_mm(a1, g1, d0, d1, b1r, W2)
    a2 = _sc_agg(*g2, sd_idx)
    return _final(a2, g2, d0, d1, b2r, bat, Wl, blr)


# speedup vs baseline: 13.5919x; 1.0460x over previous
import functools

import jax
import jax.numpy as jnp
from jax import lax
from jax.experimental import pallas as pl
from jax.experimental.pallas import tpu as pltpu
from jax.experimental.pallas import tpu_sc as plsc

N = 50000
E = 800000
VOCAB = 1000
G = 16
DE = 64
DH = 128
NCLS = 10

NP = 53248
EP = 802816
F = 32
RPT = NP // 16
RB = RPT // 128
EB_DEG = (EP // 32) // 128
EB_AGG = (EP // 16) // 128
R = 1024
GRID = NP // R

_mesh = plsc.VectorSubcoreMesh(core_axis_name="c", subcore_axis_name="s")
_sc_params = pltpu.CompilerParams(use_tc_tiling_on_sc=False)
_sc_params_nl = pltpu.CompilerParams(use_tc_tiling_on_sc=False,
                                     needs_layout_passes=False)


def _t1_body(e_ref, w_ref, o_ref):
    o_ref[...] = jnp.dot(e_ref[...], w_ref[...], preferred_element_type=jnp.float32)


def _table(emb, W1):
    return pl.pallas_call(
        _t1_body,
        out_shape=jax.ShapeDtypeStruct((VOCAB, DH), jnp.float32),
    )(emb, W1)


def _dinv(d):
    return lax.rsqrt(d + 1.0)


def _scale_body(hw_ref, d_ref, o0, o1, o2, o3):
    g = hw_ref[...] * _dinv(d_ref[...])
    o0[...] = g[:, 0:32]
    o1[...] = g[:, 32:64]
    o2[...] = g[:, 64:96]
    o3[...] = g[:, 96:128]


def _scale_chunks(hw, d):
    return pl.pallas_call(
        _scale_body,
        grid=(GRID,),
        in_specs=[
            pl.BlockSpec((R, DH), lambda i: (i, 0)),
            pl.BlockSpec((R, 1), lambda i: (i, 0)),
        ],
        out_specs=[pl.BlockSpec((R, F), lambda i: (i, 0))] * 4,
        out_shape=[jax.ShapeDtypeStruct((NP, F), jnp.float32)] * 4,
    )(hw, d)


def _post_mm_body(a0, a1, a2, a3, g0, g1, g2, g3, d_ref, b_ref,
                  w_ref, o0, o1, o2, o3):
    agg = jnp.concatenate([a0[...], a1[...], a2[...], a3[...]], axis=1)
    gg = jnp.concatenate([g0[...], g1[...], g2[...], g3[...]], axis=1)
    dinv = _dinv(d_ref[...])
    h = jnp.maximum((agg + gg) * dinv + b_ref[...], 0.0)
    g = jnp.dot(h, w_ref[...], preferred_element_type=jnp.float32) * dinv
    o0[...] = g[:, 0:32]
    o1[...] = g[:, 32:64]
    o2[...] = g[:, 64:96]
    o3[...] = g[:, 96:128]


def _post_mm(aggs, gs, d, b, W):
    return pl.pallas_call(
        _post_mm_body,
        grid=(GRID,),
        in_specs=[pl.BlockSpec((R, F), lambda i: (i, 0))] * 8 + [
            pl.BlockSpec((R, 1), lambda i: (i, 0)),
            pl.BlockSpec((1, DH), lambda i: (0, 0)),
            pl.BlockSpec((DH, DH), lambda i: (0, 0)),
        ],
        out_specs=[pl.BlockSpec((R, F), lambda i: (i, 0))] * 4,
        out_shape=[jax.ShapeDtypeStruct((NP, F), jnp.float32)] * 4,
    )(*aggs, *gs, d, b, W)


def _final_body(a0, a1, a2, a3, g0, g1, g2, g3, d_ref, b_ref,
                bat_ref, wl_ref, bl_ref, o_ref, sums, cnt):
    i = pl.program_id(0)
    agg = jnp.concatenate([a0[...], a1[...], a2[...], a3[...]], axis=1)
    gg = jnp.concatenate([g0[...], g1[...], g2[...], g3[...]], axis=1)
    h = (agg + gg) * _dinv(d_ref[...]) + b_ref[...]
    h = jnp.maximum(h, 0.0)
    oh = (bat_ref[...] == lax.broadcasted_iota(jnp.int32, (R, G), 1)
          ).astype(jnp.float32)
    dn = (((0,), (0,)), ((), ()))
    ps = lax.dot_general(oh, h, dn, preferred_element_type=jnp.float32)
    pc = lax.dot_general(oh, jnp.ones_like(h), dn,
                         preferred_element_type=jnp.float32)

    @pl.when(i == 0)
    def _init():
        sums[...] = ps
        cnt[...] = pc

    @pl.when(i > 0)
    def _acc():
        sums[...] += ps
        cnt[...] += pc

    @pl.when(i == GRID - 1)
    def _fin():
        pooled = sums[...] / jnp.maximum(cnt[...], 1.0)
        o_ref[...] = jnp.dot(pooled, wl_ref[...],
                             preferred_element_type=jnp.float32) + bl_ref[...]


def _final(aggs, gs, d, b, bat, Wl, bl):
    return pl.pallas_call(
        _final_body,
        grid=(GRID,),
        in_specs=[pl.BlockSpec((R, F), lambda i: (i, 0))] * 8 + [
            pl.BlockSpec((R, 1), lambda i: (i, 0)),
            pl.BlockSpec((1, DH), lambda i: (0, 0)),
            pl.BlockSpec((R, 1), lambda i: (i, 0)),
            pl.BlockSpec((DH, NCLS), lambda i: (0, 0)),
            pl.BlockSpec((1, NCLS), lambda i: (0, 0)),
        ],
        out_specs=pl.BlockSpec((G, NCLS), lambda i: (0, 0)),
        out_shape=jax.ShapeDtypeStruct((G, NCLS), jnp.float32),
        scratch_shapes=[
            pltpu.VMEM((G, DH), jnp.float32),
            pltpu.VMEM((G, DH), jnp.float32),
        ],
    )(*aggs, *gs, d, b, bat, Wl, bl)


@functools.partial(
    pl.kernel, mesh=_mesh,
    out_type=jax.ShapeDtypeStruct((NP, DH), jnp.float32),
    scratch_types=[
        pltpu.VMEM((13, 128), jnp.int32),
        pltpu.VMEM((4, 128, DH), jnp.float32),
        pltpu.SemaphoreType.DMA,
    ],
    compiler_params=_sc_params,
)
def _sc_gather(tab, xi, o, idx, buf, sem):
    c = lax.axis_index("c")
    s = lax.axis_index("s")
    wid = s * 2 + c
    nb = NP // 32 // 128
    row0 = wid * nb
    pltpu.sync_copy(xi.at[pl.ds(row0, nb)], idx)
    descs = [None] * nb
    for j in range(nb):
        if j >= 4:
            descs[j - 4].wait()
            pltpu.sync_copy(buf.at[(j - 4) % 4],
                            o.at[pl.ds((row0 + j - 4) * 128, 128)])
        descs[j] = pltpu.async_copy(tab.at[idx.at[j]], buf.at[j % 4], sem)
    for j in range(nb - 4, nb):
        descs[j].wait()
        pltpu.sync_copy(buf.at[j % 4], o.at[pl.ds((row0 + j) * 128, 128)])


@functools.partial(
    pl.kernel, mesh=_mesh,
    out_type=jax.ShapeDtypeStruct((2, NP // 128, 128), jnp.float32),
    scratch_types=[
        pltpu.VMEM((4, 128), jnp.int32),
        pltpu.VMEM((NP // 128, 128), jnp.float32),
        pltpu.VMEM((26, 128), jnp.float32),
        pltpu.VMEM((128,), jnp.int32),
        pltpu.VMEM((32,), jnp.int32),
        pltpu.VMEM_SHARED((NP // 128, 128), jnp.float32),
        pltpu.SemaphoreType.DMA,
    ],
    compiler_params=_sc_params_nl,
)
def _sc_deg(dst, o, idx, cnt, zb, ident, ident2, acc, sem):
    c = lax.axis_index("c")
    s = lax.axis_index("s")
    zero16 = jnp.zeros((16,), jnp.float32)
    one16 = jnp.ones((16,), jnp.float32)
    for i in range(26):
        for t in range(8):
            zb[i, pl.ds(t * 16, 16)] = zero16

    def zrow(i, carry):
        for t in range(8):
            cnt[i, pl.ds(t * 16, 16)] = zero16
        return carry

    lax.fori_loop(0, NP // 128, zrow, 0)

    pltpu.sync_copy(zb, acc.at[pl.ds(s * 26, 26)])
    plsc.subcore_barrier()

    def ebody(k, carry):
        row0 = c * (EP // 256) + s * EB_DEG + k * 4
        pltpu.sync_copy(dst.at[pl.ds(row0, 4)], idx)
        for j in range(4):
            for t in range(8):
                v = idx[j, pl.ds(t * 16, 16)]
                hi = jnp.right_shift(v, 7)
                lo = jnp.bitwise_and(v, 127)
                plsc.addupdate_scatter(cnt, [hi, lo], one16)
        return carry

    lax.fori_loop(0, EB_DEG // 4, ebody, 0)

    for k in range(3):
        for t in range(8):
            ident[pl.ds(t * 16, 16)] = (
                lax.iota(jnp.int32, 16) + (k * 128 + t * 16))
        pltpu.async_copy(cnt.at[pl.ds(k * 128, 128)],
                         acc.at[ident], sem, add=True).wait()
    for t in range(2):
        ident2[pl.ds(t * 16, 16)] = (
            lax.iota(jnp.int32, 16) + (384 + t * 16))
    pltpu.async_copy(cnt.at[pl.ds(384, 32)],
                     acc.at[ident2], sem, add=True).wait()
    plsc.subcore_barrier()
    pltpu.sync_copy(acc.at[pl.ds(s * 26, 26)], o.at[c, pl.ds(s * 26, 26)])


@functools.partial(
    pl.kernel, mesh=_mesh,
    out_type=[jax.ShapeDtypeStruct((NP, F), jnp.float32)] * 4,
    scratch_types=[
        pltpu.VMEM((8, 128), jnp.int32),
        pltpu.VMEM((4, 128, F), jnp.float32),
        pltpu.VMEM((128, F), jnp.float32),
        pltpu.VMEM_SHARED((NP, F), jnp.float32),
        pltpu.SemaphoreType.DMA,
        pltpu.SemaphoreType.DMA,
    ],
    compiler_params=_sc_params,
)
def _sc_agg(g0, g1, g2, g3, sd_idx, o0, o1, o2, o3,
            idx8, buf, zb, acc, sem_g, sem_s):
    c = lax.axis_index("c")
    s = lax.axis_index("s")
    for i in range(128):
        for j in range(F // 16):
            zb[i, pl.ds(j * 16, 16)] = jnp.zeros((16,), jnp.float32)

    def run(g, o):
        def zbody(k, carry):
            pltpu.sync_copy(zb, acc.at[pl.ds(s * RPT + k * 128, 128)])
            return carry

        lax.fori_loop(0, RB, zbody, 0)
        plsc.subcore_barrier()

        def ebody(k, carry):
            grp = s * (EB_AGG // 4) + k
            pltpu.sync_copy(sd_idx.at[grp], idx8)
            gd = [pltpu.async_copy(g.at[idx8.at[j]], buf.at[j], sem_g)
                  for j in range(4)]
            sd = [None] * 4
            for j in range(4):
                gd[j].wait()
                sd[j] = pltpu.async_copy(buf.at[j], acc.at[idx8.at[4 + j]],
                                         sem_s, add=True)
            for j in range(4):
                sd[j].wait()
            return carry

        lax.fori_loop(0, EB_AGG // 4, ebody, 0)
        plsc.subcore_barrier()
        pltpu.sync_copy(acc.at[pl.ds(s * RPT, RPT)], o.at[pl.ds(s * RPT, RPT)])
        plsc.subcore_barrier()

    @pl.when(c == 0)
    def _sc0():
        run(g0, o0)
        run(g1, o1)

    @pl.when(c == 1)
    def _sc1():
        run(g2, o2)
        run(g3, o3)


def kernel(x, edge_index, batch, emb, W1, b1, W2, b2, Wl, bl):
    x_pad = jnp.concatenate([x, jnp.zeros((NP - N,), jnp.int32)])
    x_pad = x_pad.reshape(NP // 128, 128)
    src = jnp.concatenate([edge_index[0], jnp.full((EP - E,), N, jnp.int32)])
    dst = jnp.concatenate([edge_index[1], jnp.full((EP - E,), N, jnp.int32)])
    sd_idx = jnp.concatenate([src.reshape(EP // 512, 4, 128),
                              dst.reshape(EP // 512, 4, 128)], axis=1)
    dst = dst.reshape(EP // 128, 128)
    bat = jnp.concatenate([batch, jnp.full((NP - N,), G, jnp.int32)])
    bat = bat.reshape(NP, 1)
    b1r = b1.reshape(1, DH)
    b2r = b2.reshape(1, DH)
    blr = bl.reshape(1, NCLS)

    T1 = _table(emb, W1)
    hw1 = _sc_gather(T1, x_pad)
    dg = _sc_deg(dst)
    deg = (dg[0] + dg[1]).reshape(NP, 1)

    g1 = _scale_chunks(hw1, deg)
    a1 = _sc_agg(*g1, sd_idx)
    g2 = _post_mm(a1, g1, deg, b1r, W2)
    a2 = _sc_agg(*g2, sd_idx)
    return _final(a2, g2, deg, b2r, bat, Wl, blr)
